# baseline (speedup 1.0000x reference)
import functools

import jax
import jax.numpy as jnp
from jax.experimental import pallas as pl
from jax.experimental.pallas import tpu as pltpu

_VMEM = 100 * 1024 * 1024


def _xwt_kernel(x_ref, w1t_ref, o_ref):
    o_ref[...] = jax.lax.dot_general(
        w1t_ref[...], x_ref[...].astype(jnp.bfloat16),
        (((1,), (1,)), ((), ())),
        preferred_element_type=jnp.float32).astype(o_ref.dtype)


def _fused_t_kernel(a_hbm, xwt_ref, b1c_ref, w2t_ref, b2c_ref, out_ref,
                    a_vmem, hwt_ref, acc_ref, sem, *, n_classes, tile, nt):
    i = pl.program_id(0)
    t = tile

    @pl.when(i == 0)
    def _():
        for s in range(nt):
            pltpu.make_async_copy(
                a_hbm.at[pl.ds(s * t, t), :],
                a_vmem.at[pl.ds(s * t, t), :],
                sem.at[s],
            ).start()

    pltpu.make_async_copy(
        a_hbm.at[pl.ds(i * t, t), :],
        a_vmem.at[pl.ds(i * t, t), :],
        sem.at[i],
    ).wait()

    islab = pl.ds(pl.multiple_of(i * t, t), t)

    acc1t = jax.lax.dot_general(
        xwt_ref[...], a_vmem[islab, :], (((1,), (1,)), ((), ())),
        preferred_element_type=jnp.float32)
    h = jnp.maximum(acc1t + b1c_ref[...], 0.0).astype(jnp.bfloat16)
    hwt_ref[:, islab] = jnp.dot(w2t_ref[...], h,
                                preferred_element_type=jnp.float32
                                ).astype(jnp.bfloat16)

    def row_body(c, acc):
        cs = pl.ds(pl.multiple_of(c * t, t), t)
        return acc + jax.lax.dot_general(
            hwt_ref[:, cs], a_vmem[islab, cs], (((1,), (1,)), ((), ())),
            preferred_element_type=jnp.float32)

    acc_ref[:, islab] = jax.lax.fori_loop(
        0, i + 1, row_body, jnp.zeros(acc_ref.shape[:1] + (t,), jnp.float32))

    def col_body(r, carry):
        rs = pl.ds(pl.multiple_of(r * t, t), t)
        acc_ref[:, rs] += jax.lax.dot_general(
            hwt_ref[:, islab], a_vmem[rs, islab], (((1,), (1,)), ((), ())),
            preferred_element_type=jnp.float32)
        return carry

    jax.lax.fori_loop(0, i, col_body, 0)

    @pl.when(i == nt - 1)
    def _():
        z = acc_ref[...] + b2c_ref[...]
        row = jax.lax.broadcasted_iota(jnp.int32, z.shape, 0)
        valid = row < n_classes
        z = jnp.where(valid, z, -jnp.inf)
        mx = jnp.max(z, axis=0, keepdims=True)
        s = z - mx
        lse = jnp.log(jnp.sum(jnp.exp(s), axis=0, keepdims=True))
        out_ref[...] = jnp.where(valid, s - lse, 0.0)


def kernel(a_hat, x, w1, b1, w2, b2):
    n, f = x.shape
    hidden = w1.shape[1]
    n_classes = w2.shape[1]
    cp = 8

    a_p = a_hat
    if a_p.shape != (n, n) or a_p.dtype != jnp.bfloat16:
        a_p = jnp.zeros((n, n), jnp.bfloat16).at[:n, :n].set(
            a_hat[:n, :n].astype(jnp.bfloat16))

    w1t = w1.T.astype(jnp.bfloat16)
    b1c = b1.astype(jnp.float32).reshape(hidden, 1)
    w2t = jnp.zeros((cp, hidden), jnp.bfloat16).at[:n_classes, :].set(
        w2.T.astype(jnp.bfloat16))
    b2c = jnp.zeros((cp, 1), jnp.float32).at[:n_classes, 0].set(
        b2.astype(jnp.float32))

    tile = min(512, n)
    nt = n // tile

    xwt = pl.pallas_call(
        _xwt_kernel,
        out_shape=jax.ShapeDtypeStruct((hidden, n), jnp.bfloat16),
        grid=(nt,),
        in_specs=[pl.BlockSpec((tile, f), lambda i: (i, 0)),
                  pl.BlockSpec((hidden, f), lambda i: (0, 0))],
        out_specs=pl.BlockSpec((hidden, tile), lambda i: (0, i)),
        compiler_params=pltpu.CompilerParams(
            dimension_semantics=("parallel",),
            vmem_limit_bytes=_VMEM,
        ),
    )(x, w1t)

    out_t = pl.pallas_call(
        functools.partial(_fused_t_kernel, n_classes=n_classes, tile=tile,
                          nt=nt),
        out_shape=jax.ShapeDtypeStruct((cp, n), jnp.float32),
        grid=(nt,),
        in_specs=[pl.BlockSpec(memory_space=pl.ANY),
                  pl.BlockSpec((hidden, n), lambda i: (0, 0)),
                  pl.BlockSpec((hidden, 1), lambda i: (0, 0)),
                  pl.BlockSpec((cp, hidden), lambda i: (0, 0)),
                  pl.BlockSpec((cp, 1), lambda i: (0, 0))],
        out_specs=pl.BlockSpec((cp, n), lambda i: (0, 0)),
        scratch_shapes=[
            pltpu.VMEM((n, n), jnp.bfloat16),
            pltpu.VMEM((cp, n), jnp.bfloat16),
            pltpu.VMEM((cp, n), jnp.float32),
            pltpu.SemaphoreType.DMA((nt,)),
        ],
        compiler_params=pltpu.CompilerParams(
            dimension_semantics=("arbitrary",),
            vmem_limit_bytes=_VMEM,
        ),
    )(a_p, xwt, b1c, w2t, b2c)

    return out_t[:n_classes, :].T


# speedup vs baseline: 1.0617x; 1.0617x over previous
import functools

import jax
import jax.numpy as jnp
from jax.experimental import pallas as pl
from jax.experimental.pallas import tpu as pltpu

_VMEM = 61 * 1024 * 1024


def _mega_kernel(x_hbm, a_hbm, w1t_ref, b1c_ref, w2t_ref, b2c_ref, out_ref,
                 x_vmem, a_vmem, xwt_ref, hwt_ref, semx, sema,
                 *, n_classes, tile, nt):
    p = pl.program_id(0)
    i = pl.program_id(1)
    t = tile

    @pl.when((p == 0) & (i == 0))
    def _():
        for s in range(nt):
            pltpu.make_async_copy(
                x_hbm.at[pl.ds(s * t, t), :],
                x_vmem.at[pl.ds(s * t, t), :],
                semx.at[s]).start()
        for s in range(nt):
            pltpu.make_async_copy(
                a_hbm.at[pl.ds(s * t, t), :],
                a_vmem.at[pl.ds(s * t, t), :],
                sema.at[s]).start()

    islab = pl.ds(pl.multiple_of(i * t, t), t)

    @pl.when(p == 0)
    def _():
        pltpu.make_async_copy(
            x_hbm.at[pl.ds(i * t, t), :],
            x_vmem.at[pl.ds(i * t, t), :],
            semx.at[i]).wait()
        xb = x_vmem[islab, :].astype(jnp.bfloat16)
        xwt_ref[:, islab] = jax.lax.dot_general(
            w1t_ref[...], xb, (((1,), (1,)), ((), ())),
            preferred_element_type=jnp.float32).astype(jnp.bfloat16)

    @pl.when(p == 1)
    def _():
        pltpu.make_async_copy(
            a_hbm.at[pl.ds(i * t, t), :],
            a_vmem.at[pl.ds(i * t, t), :],
            sema.at[i]).wait()
        acc1t = jax.lax.dot_general(
            xwt_ref[...], a_vmem[islab, :], (((1,), (1,)), ((), ())),
            preferred_element_type=jnp.float32)
        h = jnp.maximum(acc1t + b1c_ref[...], 0.0).astype(jnp.bfloat16)
        hwt_ref[:, islab] = jnp.dot(
            w2t_ref[...], h, preferred_element_type=jnp.float32
            ).astype(jnp.bfloat16)

    @pl.when(p == 2)
    def _():
        z = jax.lax.dot_general(
            hwt_ref[...], a_vmem[islab, :], (((1,), (1,)), ((), ())),
            preferred_element_type=jnp.float32) + b2c_ref[...]
        row = jax.lax.broadcasted_iota(jnp.int32, z.shape, 0)
        valid = row < n_classes
        z = jnp.where(valid, z, -jnp.inf)
        mx = jnp.max(z, axis=0, keepdims=True)
        s = z - mx
        lse = jnp.log(jnp.sum(jnp.exp(s), axis=0, keepdims=True))
        out_ref[:, islab] = jnp.where(valid, s - lse, 0.0)


def kernel(a_hat, x, w1, b1, w2, b2):
    n, f = x.shape
    hidden = w1.shape[1]
    n_classes = w2.shape[1]
    cp = 8

    a_p = a_hat
    if a_p.shape != (n, n) or a_p.dtype != jnp.bfloat16:
        a_p = jnp.zeros((n, n), jnp.bfloat16).at[:n, :n].set(
            a_hat[:n, :n].astype(jnp.bfloat16))

    w1t = w1.T.astype(jnp.bfloat16)
    b1c = b1.astype(jnp.float32).reshape(hidden, 1)
    w2t = jnp.zeros((cp, hidden), jnp.bfloat16).at[:n_classes, :].set(
        w2.T.astype(jnp.bfloat16))
    b2c = jnp.zeros((cp, 1), jnp.float32).at[:n_classes, 0].set(
        b2.astype(jnp.float32))

    tile = min(512, n)
    nt = n // tile

    out_t = pl.pallas_call(
        functools.partial(_mega_kernel, n_classes=n_classes, tile=tile,
                          nt=nt),
        out_shape=jax.ShapeDtypeStruct((cp, n), jnp.float32),
        grid=(3, nt),
        in_specs=[pl.BlockSpec(memory_space=pl.ANY),
                  pl.BlockSpec(memory_space=pl.ANY),
                  pl.BlockSpec((hidden, f), lambda p, i: (0, 0)),
                  pl.BlockSpec((hidden, 1), lambda p, i: (0, 0)),
                  pl.BlockSpec((cp, hidden), lambda p, i: (0, 0)),
                  pl.BlockSpec((cp, 1), lambda p, i: (0, 0))],
        out_specs=pl.BlockSpec((cp, n), lambda p, i: (0, 0)),
        scratch_shapes=[
            pltpu.VMEM((n, f), jnp.float32),
            pltpu.VMEM((n, n), jnp.bfloat16),
            pltpu.VMEM((hidden, n), jnp.bfloat16),
            pltpu.VMEM((cp, n), jnp.bfloat16),
            pltpu.SemaphoreType.DMA((nt,)),
            pltpu.SemaphoreType.DMA((nt,)),
        ],
        compiler_params=pltpu.CompilerParams(
            dimension_semantics=("arbitrary", "arbitrary"),
            vmem_limit_bytes=_VMEM,
        ),
    )(x, a_p, w1t, b1c, w2t, b2c)

    return out_t[:n_classes, :].T


# speedup vs baseline: 1.1237x; 1.0583x over previous
import functools

import jax
import jax.numpy as jnp
from jax.experimental import pallas as pl
from jax.experimental.pallas import tpu as pltpu

_VMEM = 61 * 1024 * 1024


def _mega_kernel(x_hbm, a_hbm, w1t_ref, b1c_ref, w2t_ref, b2c_ref, out_ref,
                 x_vmem, a_vmem, xwt_ref, hwt_ref, semx, sema,
                 *, n_classes, tile, nt):
    p = pl.program_id(0)
    i = pl.program_id(1)
    t = tile

    @pl.when((p == 0) & (i == 0))
    def _():
        for s in range(nt):
            pltpu.make_async_copy(
                x_hbm.at[pl.ds(s * t, t), :],
                x_vmem.at[pl.ds(s * t, t), :],
                semx.at[s]).start()
        for s in range(nt):
            pltpu.make_async_copy(
                a_hbm.at[pl.ds(s * t, t), :],
                a_vmem.at[pl.ds(s * t, t), :],
                sema.at[s]).start()

    islab = pl.ds(pl.multiple_of(i * t, t), t)

    @pl.when(p == 0)
    def _():
        pltpu.make_async_copy(
            x_hbm.at[pl.ds(i * t, t), :],
            x_vmem.at[pl.ds(i * t, t), :],
            semx.at[i]).wait()
        xb = x_vmem[islab, :].astype(jnp.bfloat16)
        xwt_ref[:, islab] = jax.lax.dot_general(
            w1t_ref[...], xb, (((1,), (1,)), ((), ())),
            preferred_element_type=jnp.float32).astype(jnp.bfloat16)

    @pl.when(p == 1)
    def _():
        pltpu.make_async_copy(
            a_hbm.at[pl.ds(i * t, t), :],
            a_vmem.at[pl.ds(i * t, t), :],
            sema.at[i]).wait()
        acc1t = jax.lax.dot_general(
            xwt_ref[...], a_vmem[islab, :], (((1,), (1,)), ((), ())),
            preferred_element_type=jnp.float32)
        h = jnp.maximum(acc1t + b1c_ref[...], 0.0).astype(jnp.bfloat16)
        hwt_ref[:, islab] = jnp.dot(
            w2t_ref[...], h, preferred_element_type=jnp.float32
            ).astype(jnp.bfloat16)

    @pl.when(p == 2)
    def _():
        z = jax.lax.dot_general(
            hwt_ref[...], a_vmem[islab, :], (((1,), (1,)), ((), ())),
            preferred_element_type=jnp.float32) + b2c_ref[...]
        row = jax.lax.broadcasted_iota(jnp.int32, z.shape, 0)
        valid = row < n_classes
        z = jnp.where(valid, z, -jnp.inf)
        mx = jnp.max(z, axis=0, keepdims=True)
        s = z - mx
        lse = jnp.log(jnp.sum(jnp.exp(s), axis=0, keepdims=True))
        out_ref[:, islab] = s - lse


def kernel(a_hat, x, w1, b1, w2, b2):
    n, f = x.shape
    hidden = w1.shape[1]
    n_classes = w2.shape[1]
    cp = 8

    a_p = a_hat
    if a_p.shape != (n, n) or a_p.dtype != jnp.bfloat16:
        a_p = jnp.zeros((n, n), jnp.bfloat16).at[:n, :n].set(
            a_hat[:n, :n].astype(jnp.bfloat16))

    w1t = w1.T.astype(jnp.bfloat16)
    b1c = b1.astype(jnp.float32).reshape(hidden, 1)
    w2t = jnp.zeros((cp, hidden), jnp.bfloat16).at[:n_classes, :].set(
        w2.T.astype(jnp.bfloat16))
    b2c = jnp.zeros((cp, 1), jnp.float32).at[:n_classes, 0].set(
        b2.astype(jnp.float32))

    tile = min(1024, n)
    nt = n // tile

    out_t = pl.pallas_call(
        functools.partial(_mega_kernel, n_classes=n_classes, tile=tile,
                          nt=nt),
        out_shape=jax.ShapeDtypeStruct((cp, n), jnp.float32),
        grid=(3, nt),
        in_specs=[pl.BlockSpec(memory_space=pl.ANY),
                  pl.BlockSpec(memory_space=pl.ANY),
                  pl.BlockSpec((hidden, f), lambda p, i: (0, 0)),
                  pl.BlockSpec((hidden, 1), lambda p, i: (0, 0)),
                  pl.BlockSpec((cp, hidden), lambda p, i: (0, 0)),
                  pl.BlockSpec((cp, 1), lambda p, i: (0, 0))],
        out_specs=pl.BlockSpec((cp, n), lambda p, i: (0, 0)),
        scratch_shapes=[
            pltpu.VMEM((n, f), jnp.float32),
            pltpu.VMEM((n, n), jnp.bfloat16),
            pltpu.VMEM((hidden, n), jnp.bfloat16),
            pltpu.VMEM((cp, n), jnp.bfloat16),
            pltpu.SemaphoreType.DMA((nt,)),
            pltpu.SemaphoreType.DMA((nt,)),
        ],
        compiler_params=pltpu.CompilerParams(
            dimension_semantics=("arbitrary", "arbitrary"),
            vmem_limit_bytes=_VMEM,
        ),
    )(x, a_p, w1t, b1c, w2t, b2c)

    return out_t[:n_classes, :].T


# speedup vs baseline: 1.1305x; 1.0061x over previous
import functools

import jax
import jax.numpy as jnp
from jax.experimental import pallas as pl
from jax.experimental.pallas import tpu as pltpu

_VMEM = 61 * 1024 * 1024


def _mega_kernel(x_hbm, a_hbm, w1t_ref, b1c_ref, w2t_ref, b2c_ref, out_ref,
                 x_vmem, a_vmem, xwt_ref, hwt_ref, semx, sema,
                 *, n_classes, tile, nt):
    p = pl.program_id(0)
    i = pl.program_id(1)
    t = tile

    @pl.when((p == 0) & (i == 0))
    def _():
        for s in range(nt):
            pltpu.make_async_copy(
                x_hbm.at[pl.ds(s * t, t), :],
                x_vmem.at[pl.ds(s * t, t), :],
                semx.at[s]).start()
        for s in range(nt):
            pltpu.make_async_copy(
                a_hbm.at[pl.ds(s * t, t), :],
                a_vmem.at[pl.ds(s * t, t), :],
                sema.at[s]).start()

    islab = pl.ds(pl.multiple_of(i * t, t), t)

    @pl.when(p == 0)
    def _():
        pltpu.make_async_copy(
            x_hbm.at[pl.ds(i * t, t), :],
            x_vmem.at[pl.ds(i * t, t), :],
            semx.at[i]).wait()
        xb = x_vmem[islab, :].astype(jnp.bfloat16)
        xwt_ref[:, islab] = jax.lax.dot_general(
            w1t_ref[...], xb, (((1,), (1,)), ((), ())),
            preferred_element_type=jnp.float32).astype(jnp.bfloat16)

    @pl.when(p == 1)
    def _():
        pltpu.make_async_copy(
            a_hbm.at[pl.ds(i * t, t), :],
            a_vmem.at[pl.ds(i * t, t), :],
            sema.at[i]).wait()
        acc1t = jax.lax.dot_general(
            xwt_ref[...], a_vmem[islab, :], (((1,), (1,)), ((), ())),
            preferred_element_type=jnp.float32)
        h = jnp.maximum(acc1t + b1c_ref[...], 0.0).astype(jnp.bfloat16)
        hwt_ref[:, islab] = jnp.dot(
            w2t_ref[...], h, preferred_element_type=jnp.float32
            ).astype(jnp.bfloat16)

    @pl.when(p == 2)
    def _():
        z = jax.lax.dot_general(
            hwt_ref[...], a_vmem[islab, :], (((1,), (1,)), ((), ())),
            preferred_element_type=jnp.float32) + b2c_ref[...]
        row = jax.lax.broadcasted_iota(jnp.int32, z.shape, 0)
        valid = row < n_classes
        z = jnp.where(valid, z, -jnp.inf)
        mx = jnp.max(z, axis=0, keepdims=True)
        s = z - mx
        lse = jnp.log(jnp.sum(jnp.exp(s), axis=0, keepdims=True))
        out_ref[:, islab] = s - lse


def kernel(a_hat, x, w1, b1, w2, b2):
    n, f = x.shape
    hidden = w1.shape[1]
    n_classes = w2.shape[1]
    cp = 8

    a_p = a_hat
    if a_p.shape != (n, n) or a_p.dtype != jnp.bfloat16:
        a_p = jnp.zeros((n, n), jnp.bfloat16).at[:n, :n].set(
            a_hat[:n, :n].astype(jnp.bfloat16))

    w1t = w1.T.astype(jnp.bfloat16)
    b1c = b1.astype(jnp.float32).reshape(hidden, 1)
    w2t = jnp.zeros((cp, hidden), jnp.bfloat16).at[:n_classes, :].set(
        w2.T.astype(jnp.bfloat16))
    b2c = jnp.zeros((cp, 1), jnp.float32).at[:n_classes, 0].set(
        b2.astype(jnp.float32))

    tile = n
    nt = n // tile

    out_t = pl.pallas_call(
        functools.partial(_mega_kernel, n_classes=n_classes, tile=tile,
                          nt=nt),
        out_shape=jax.ShapeDtypeStruct((cp, n), jnp.float32),
        grid=(3, nt),
        in_specs=[pl.BlockSpec(memory_space=pl.ANY),
                  pl.BlockSpec(memory_space=pl.ANY),
                  pl.BlockSpec((hidden, f), lambda p, i: (0, 0)),
                  pl.BlockSpec((hidden, 1), lambda p, i: (0, 0)),
                  pl.BlockSpec((cp, hidden), lambda p, i: (0, 0)),
                  pl.BlockSpec((cp, 1), lambda p, i: (0, 0))],
        out_specs=pl.BlockSpec((cp, n), lambda p, i: (0, 0)),
        scratch_shapes=[
            pltpu.VMEM((n, f), jnp.float32),
            pltpu.VMEM((n, n), jnp.bfloat16),
            pltpu.VMEM((hidden, n), jnp.bfloat16),
            pltpu.VMEM((cp, n), jnp.bfloat16),
            pltpu.SemaphoreType.DMA((nt,)),
            pltpu.SemaphoreType.DMA((nt,)),
        ],
        compiler_params=pltpu.CompilerParams(
            dimension_semantics=("arbitrary", "arbitrary"),
            vmem_limit_bytes=_VMEM,
        ),
    )(x, a_p, w1t, b1c, w2t, b2c)

    return out_t[:n_classes, :].T
